# baseline (speedup 1.0000x reference)
import functools

import jax
import jax.numpy as jnp
from jax import lax
from jax.experimental import pallas as pl
from jax.experimental.pallas import tpu as pltpu
from jax.experimental.pallas import tpu_sc as plsc

_NBINS = 10
_NGROUPS = 4
_SLOTS = 12
_HW = _NGROUPS * _SLOTS
_LANES = 16
_NW = 32
_CHUNK = 8192
_UNROLL = 4


def _sc_hist_kernel(nper, acts_hbm, labels_hbm, out_cnt, out_fr,
                    abuf, lbuf, cnt, fr, sem_a, sem_l):
    wid = lax.axis_index("s") * 2 + lax.axis_index("c")
    base = wid * nper
    nchunks = nper // _CHUNK

    lanebase = lax.iota(jnp.int32, _LANES) * _HW
    ones = jnp.ones((_LANES,), jnp.float32)
    zero16 = jnp.zeros((_LANES,), jnp.float32)
    for i in range(_HW):
        cnt[pl.ds(i * _LANES, _LANES)] = zero16
        fr[pl.ds(i * _LANES, _LANES)] = zero16

    def issue(ci, buf):
        off = base + ci * _CHUNK
        for k in range(_CHUNK // 128):
            pltpu.async_copy(acts_hbm.at[pl.ds(off + k * 128, 128)],
                             abuf.at[buf, pl.ds(k * 128, 128)], sem_a)
        for k in range(_CHUNK // 128):
            pltpu.async_copy(labels_hbm.at[pl.ds(off + k * 128, 128)],
                             lbuf.at[buf, pl.ds(k * 128, 128)], sem_l)

    def wait_copy(ci, buf):
        off = base + ci * _CHUNK
        pltpu.make_async_copy(acts_hbm.at[pl.ds(off, _CHUNK)],
                              abuf.at[buf], sem_a).wait()
        pltpu.make_async_copy(labels_hbm.at[pl.ds(off, _CHUNK)],
                              lbuf.at[buf], sem_l).wait()

    def consume(buf):
        @plsc.parallel_loop(0, _CHUNK // _LANES, 1, unroll=_UNROLL)
        def _vec_body(vi):
            off = vi * _LANES
            x = abuf[buf, pl.ds(off, _LANES)]
            g = lbuf[buf, pl.ds(off, _LANES)]
            e = jnp.exp(x)
            tp = 10.999 - 1.0 / (0.1 + 0.1 * e)
            jp = tp.astype(jnp.int32)
            idx = g * _SLOTS + jp + lanebase
            plsc.addupdate_scatter(cnt, [idx], ones)
            plsc.addupdate_scatter(fr, [idx], tp)

    issue(0, 0)
    issue(1, 1)

    def chunk_body(ci, _):
        buf = lax.rem(ci, 3)

        @pl.when(ci + 2 < nchunks)
        def _prefetch():
            issue(ci + 2, lax.rem(ci + 2, 3))

        wait_copy(ci, buf)
        consume(buf)
        return 0

    lax.fori_loop(0, nchunks, chunk_body, 0)

    for ref in (cnt, fr):
        for step in (8, 4, 2, 1):
            for l in range(step):
                for v in range(_HW // _LANES):
                    a = l * _HW + v * _LANES
                    b = (l + step) * _HW + v * _LANES
                    ref[pl.ds(a, _LANES)] = (ref[pl.ds(a, _LANES)]
                                             + ref[pl.ds(b, _LANES)])

    pltpu.sync_copy(cnt.at[pl.ds(0, _HW)], out_cnt.at[pl.ds(wid * _HW, _HW)])
    pltpu.sync_copy(fr.at[pl.ds(0, _HW)], out_fr.at[pl.ds(wid * _HW, _HW)])


def _sc_finish_kernel(cnt_hbm, fr_hbm, out_hbm, cbuf, fbuf, aab, obuf, semf):
    wid = lax.axis_index("s") * 2 + lax.axis_index("c")

    @pl.when(wid == 0)
    def _work():
        a1 = pltpu.async_copy(cnt_hbm, cbuf, semf)
        a2 = pltpu.async_copy(fr_hbm, fbuf, semf)
        a1.wait()
        a2.wait()
        lane = lax.iota(jnp.int32, _LANES)
        lanef = lane.astype(jnp.float32)

        for v in range(_HW // _LANES):
            c = cbuf[pl.ds(v * _LANES, _LANES)]
            f = fbuf[pl.ds(v * _LANES, _LANES)]
            for w in range(1, _NW):
                c = c + cbuf[pl.ds(w * _HW + v * _LANES, _LANES)]
                f = f + fbuf[pl.ds(w * _HW + v * _LANES, _LANES)]
            jp = lax.rem(lane + v * _LANES, jnp.full((_LANES,), _SLOTS,
                                                     jnp.int32))
            f = f - jp.astype(jnp.float32) * c
            cbuf[pl.ds(v * _LANES, _LANES)] = c - f
            fbuf[pl.ds(v * _LANES, _LANES)] = f

        m10 = lane < 10
        for g in range(_NGROUPS):
            hi = cbuf[pl.ds(g * _SLOTS + 1, _LANES)]
            lo = fbuf[pl.ds(g * _SLOTS, _LANES)]
            row = jnp.where(m10, 0.1 * (hi + lo) + 1e-4, 0.0)
            row = row / jnp.broadcast_to(jnp.sum(row), (_LANES,))
            row = row / jnp.broadcast_to(jnp.sum(row), (_LANES,))
            aab[pl.ds(g * _LANES, _LANES)] = row

        m4 = lane < 4
        rowbase = jnp.where(m4, lane * _LANES, 0)

        def body(i, st):
            idxf, obj = st
            idxi = idxf.astype(jnp.int32)
            vals = plsc.load_gather(aab, [rowbase + idxi])
            valsm = jnp.where(m4, vals, 1e30)
            mx = jnp.broadcast_to(jnp.max(jnp.where(m4, idxf, -1.0)),
                                  (_LANES,))
            mn = jnp.broadcast_to(jnp.min(jnp.where(m4, idxf, 1e9)),
                                  (_LANES,))
            active = mx < float(_NBINS)
            minval = jnp.broadcast_to(jnp.min(valsm), (_LANES,))
            ind = jnp.broadcast_to(
                jnp.min(jnp.where(valsm == minval, lanef, 99.0)), (_LANES,))
            obj = obj + jnp.where(active & (lane == 0),
                                  (mx - mn) * minval, 0.0)
            plsc.addupdate_scatter(aab, [rowbase + idxi], -minval,
                                   mask=m4 & active)
            idxf = idxf + jnp.where((lanef == ind) & active, 1.0, 0.0)
            return idxf, obj

        zero16 = jnp.zeros((_LANES,), jnp.float32)
        _, obj = lax.fori_loop(0, 37, body, (zero16, zero16))
        obuf[...] = obj
        pltpu.sync_copy(obuf, out_hbm)


def kernel(acts, group_labels):
    n = acts.shape[0]
    nper = n // _NW

    mesh = plsc.VectorSubcoreMesh(core_axis_name="c", subcore_axis_name="s")
    sc_hist = pl.kernel(
        functools.partial(_sc_hist_kernel, nper),
        mesh=mesh,
        out_type=(jax.ShapeDtypeStruct((_NW * _HW,), jnp.float32),
                  jax.ShapeDtypeStruct((_NW * _HW,), jnp.float32)),
        scratch_types=[
            pltpu.VMEM((3, _CHUNK), jnp.float32),
            pltpu.VMEM((3, _CHUNK), jnp.int32),
            pltpu.VMEM((_LANES * _HW,), jnp.float32),
            pltpu.VMEM((_LANES * _HW,), jnp.float32),
            pltpu.SemaphoreType.DMA,
            pltpu.SemaphoreType.DMA,
        ],
        compiler_params=pltpu.CompilerParams(needs_layout_passes=False),
    )
    pcnt, pfr = sc_hist(acts, group_labels)

    sc_finish = pl.kernel(
        _sc_finish_kernel,
        mesh=mesh,
        out_type=jax.ShapeDtypeStruct((_LANES,), jnp.float32),
        scratch_types=[
            pltpu.VMEM((_NW * _HW,), jnp.float32),
            pltpu.VMEM((_NW * _HW,), jnp.float32),
            pltpu.VMEM((_NGROUPS * _LANES,), jnp.float32),
            pltpu.VMEM((_LANES,), jnp.float32),
            pltpu.SemaphoreType.DMA,
        ],
        compiler_params=pltpu.CompilerParams(needs_layout_passes=False),
    )
    obj = sc_finish(pcnt, pfr)
    return obj[0].reshape(())


# speedup vs baseline: 1.0077x; 1.0077x over previous
import functools

import jax
import jax.numpy as jnp
from jax import lax
from jax.experimental import pallas as pl
from jax.experimental.pallas import tpu as pltpu
from jax.experimental.pallas import tpu_sc as plsc

_NBINS = 10
_NGROUPS = 4
_SLOTS = 12
_HW = _NGROUPS * _SLOTS
_LANES = 16
_NW = 32
_CHUNK = 8192
_UNROLL = 4


def _sc_hist_kernel(nper, acts_hbm, labels_hbm, out_cnt, out_fr,
                    abuf, lbuf, cnt, fr, sem_a, sem_l):
    wid = lax.axis_index("s") * 2 + lax.axis_index("c")
    base = wid * nper
    nchunks = nper // _CHUNK

    lanebase = lax.iota(jnp.int32, _LANES) * _HW
    ones = jnp.ones((_LANES,), jnp.float32)
    zero16 = jnp.zeros((_LANES,), jnp.float32)
    for i in range(_HW):
        cnt[pl.ds(i * _LANES, _LANES)] = zero16
        fr[pl.ds(i * _LANES, _LANES)] = zero16

    def issue(ci, buf):
        off = base + ci * _CHUNK
        for k in range(_CHUNK // 128):
            pltpu.async_copy(acts_hbm.at[pl.ds(off + k * 128, 128)],
                             abuf.at[buf, pl.ds(k * 128, 128)], sem_a)
        for k in range(_CHUNK // 128):
            pltpu.async_copy(labels_hbm.at[pl.ds(off + k * 128, 128)],
                             lbuf.at[buf, pl.ds(k * 128, 128)], sem_l)

    def wait_copy(ci, buf):
        off = base + ci * _CHUNK
        pltpu.make_async_copy(acts_hbm.at[pl.ds(off, _CHUNK)],
                              abuf.at[buf], sem_a).wait()
        pltpu.make_async_copy(labels_hbm.at[pl.ds(off, _CHUNK)],
                              lbuf.at[buf], sem_l).wait()

    def consume(buf):
        @plsc.parallel_loop(0, _CHUNK // _LANES, 1, unroll=_UNROLL)
        def _vec_body(vi):
            off = vi * _LANES
            x = abuf[buf, pl.ds(off, _LANES)]
            g = lbuf[buf, pl.ds(off, _LANES)]
            e = jnp.exp(x)
            tp = 10.999 - 1.0 / (0.1 + 0.1 * e)
            jp = tp.astype(jnp.int32)
            idx = g * _SLOTS + jp + lanebase
            plsc.addupdate_scatter(cnt, [idx], ones)
            plsc.addupdate_scatter(fr, [idx], tp)

    issue(0, 0)

    def chunk_body(ci, _):
        buf = lax.rem(ci, 2)

        @pl.when(ci + 1 < nchunks)
        def _prefetch():
            issue(ci + 1, 1 - buf)

        wait_copy(ci, buf)
        consume(buf)
        return 0

    lax.fori_loop(0, nchunks, chunk_body, 0)

    for ref in (cnt, fr):
        for step in (8, 4, 2, 1):
            for l in range(step):
                for v in range(_HW // _LANES):
                    a = l * _HW + v * _LANES
                    b = (l + step) * _HW + v * _LANES
                    ref[pl.ds(a, _LANES)] = (ref[pl.ds(a, _LANES)]
                                             + ref[pl.ds(b, _LANES)])

    pltpu.sync_copy(cnt.at[pl.ds(0, _HW)], out_cnt.at[pl.ds(wid * _HW, _HW)])
    pltpu.sync_copy(fr.at[pl.ds(0, _HW)], out_fr.at[pl.ds(wid * _HW, _HW)])


def _sc_finish_kernel(cnt_hbm, fr_hbm, out_hbm, cbuf, fbuf, aab, obuf, semf):
    wid = lax.axis_index("s") * 2 + lax.axis_index("c")

    @pl.when(wid == 0)
    def _work():
        a1 = pltpu.async_copy(cnt_hbm, cbuf, semf)
        a2 = pltpu.async_copy(fr_hbm, fbuf, semf)
        a1.wait()
        a2.wait()
        lane = lax.iota(jnp.int32, _LANES)
        lanef = lane.astype(jnp.float32)

        for v in range(_HW // _LANES):
            c = cbuf[pl.ds(v * _LANES, _LANES)]
            f = fbuf[pl.ds(v * _LANES, _LANES)]
            for w in range(1, _NW):
                c = c + cbuf[pl.ds(w * _HW + v * _LANES, _LANES)]
                f = f + fbuf[pl.ds(w * _HW + v * _LANES, _LANES)]
            jp = lax.rem(lane + v * _LANES, jnp.full((_LANES,), _SLOTS,
                                                     jnp.int32))
            f = f - jp.astype(jnp.float32) * c
            cbuf[pl.ds(v * _LANES, _LANES)] = c - f
            fbuf[pl.ds(v * _LANES, _LANES)] = f

        m10 = lane < 10
        for g in range(_NGROUPS):
            hi = cbuf[pl.ds(g * _SLOTS + 1, _LANES)]
            lo = fbuf[pl.ds(g * _SLOTS, _LANES)]
            row = jnp.where(m10, 0.1 * (hi + lo) + 1e-4, 0.0)
            row = row / jnp.broadcast_to(jnp.sum(row), (_LANES,))
            row = row / jnp.broadcast_to(jnp.sum(row), (_LANES,))
            aab[pl.ds(g * _LANES, _LANES)] = row

        m4 = lane < 4
        rowbase = jnp.where(m4, lane * _LANES, 0)

        def body(i, st):
            idxf, obj = st
            idxi = idxf.astype(jnp.int32)
            vals = plsc.load_gather(aab, [rowbase + idxi])
            valsm = jnp.where(m4, vals, 1e30)
            mx = jnp.broadcast_to(jnp.max(jnp.where(m4, idxf, -1.0)),
                                  (_LANES,))
            mn = jnp.broadcast_to(jnp.min(jnp.where(m4, idxf, 1e9)),
                                  (_LANES,))
            active = mx < float(_NBINS)
            minval = jnp.broadcast_to(jnp.min(valsm), (_LANES,))
            ind = jnp.broadcast_to(
                jnp.min(jnp.where(valsm == minval, lanef, 99.0)), (_LANES,))
            obj = obj + jnp.where(active & (lane == 0),
                                  (mx - mn) * minval, 0.0)
            plsc.addupdate_scatter(aab, [rowbase + idxi], -minval,
                                   mask=m4 & active)
            idxf = idxf + jnp.where((lanef == ind) & active, 1.0, 0.0)
            return idxf, obj

        zero16 = jnp.zeros((_LANES,), jnp.float32)
        _, obj = lax.fori_loop(0, 37, body, (zero16, zero16))
        obuf[...] = obj
        pltpu.sync_copy(obuf, out_hbm)


def kernel(acts, group_labels):
    n = acts.shape[0]
    nper = n // _NW

    mesh = plsc.VectorSubcoreMesh(core_axis_name="c", subcore_axis_name="s")
    sc_hist = pl.kernel(
        functools.partial(_sc_hist_kernel, nper),
        mesh=mesh,
        out_type=(jax.ShapeDtypeStruct((_NW * _HW,), jnp.float32),
                  jax.ShapeDtypeStruct((_NW * _HW,), jnp.float32)),
        scratch_types=[
            pltpu.VMEM((2, _CHUNK), jnp.float32),
            pltpu.VMEM((2, _CHUNK), jnp.int32),
            pltpu.VMEM((_LANES * _HW,), jnp.float32),
            pltpu.VMEM((_LANES * _HW,), jnp.float32),
            pltpu.SemaphoreType.DMA,
            pltpu.SemaphoreType.DMA,
        ],
        compiler_params=pltpu.CompilerParams(needs_layout_passes=False),
    )
    pcnt, pfr = sc_hist(acts, group_labels)

    sc_finish = pl.kernel(
        _sc_finish_kernel,
        mesh=mesh,
        out_type=jax.ShapeDtypeStruct((_LANES,), jnp.float32),
        scratch_types=[
            pltpu.VMEM((_NW * _HW,), jnp.float32),
            pltpu.VMEM((_NW * _HW,), jnp.float32),
            pltpu.VMEM((_NGROUPS * _LANES,), jnp.float32),
            pltpu.VMEM((_LANES,), jnp.float32),
            pltpu.SemaphoreType.DMA,
        ],
        compiler_params=pltpu.CompilerParams(needs_layout_passes=False),
    )
    obj = sc_finish(pcnt, pfr)
    return obj[0].reshape(())
